# baseline (speedup 1.0000x reference)
import jax
import jax.numpy as jnp
from jax import lax
from jax.experimental import pallas as pl
from jax.experimental.pallas import tpu as pltpu
from jax.experimental.pallas import tpu_sc as plsc

_LANES = 16


def _sc_segmax(x_a, ptr_a, *, n, d, seg_per_w, chunk,
               n_workers, tw, nseg_real, nbuf):
    nseg_pad = n_workers * seg_per_w
    kg = d // _LANES
    mesh = plsc.VectorSubcoreMesh(core_axis_name="c", subcore_axis_name="s")
    num_cores = 2
    bsteps = max(1, (tw - 1).bit_length())

    def pscal(ref, i):
        return ref[pl.ds(i, _LANES)][0]

    def body(x_hbm, p_hbm, m_hbm, ptr_v, m_v, his_v, ids_v, semo,
             *bufs_sems):
        bufs = bufs_sems[:nbuf]
        sems = bufs_sems[nbuf:]
        wid = lax.axis_index("s") * num_cores + lax.axis_index("c")
        s0 = pl.multiple_of(wid * seg_per_w, 8)
        neg = jnp.full((_LANES,), -jnp.inf, jnp.float32)
        zero = jnp.zeros((_LANES,), jnp.float32)
        zlim = jnp.clip(nseg_real - wid * seg_per_w, 0, seg_per_w)

        def reduce_rows(buf_v, base, g0, g1, acc):
            def rbody(g, a):
                idx = g - base
                return tuple(
                    jnp.maximum(a[q], buf_v[idx, pl.ds(_LANES * q, _LANES)])
                    for q in range(kg))
            return lax.fori_loop(g0, g1, rbody, acc)

        if True:
            pltpu.sync_copy(p_hbm.at[pl.ds(s0, seg_per_w + 24)], ptr_v)
            rlo = pscal(ptr_v, 0)
            rhi = pscal(ptr_v, seg_per_w)
            nch = jnp.maximum((rhi - rlo + chunk - 1) // chunk, 1)

            def chunk_base(c):
                clo = rlo + c * chunk
                return pl.multiple_of(
                    jnp.minimum(clo - clo % 8, n - (chunk + 8)), 8)

            def dma_start(c, buf, sem):
                @pl.when(c < nch)
                def _():
                    pltpu.make_async_copy(
                        x_hbm.at[pl.ds(chunk_base(c), chunk + 8), :], buf,
                        sem).start()

            def dma_wait(c, buf, sem):
                @pl.when(c < nch)
                def _():
                    pltpu.make_async_copy(
                        x_hbm.at[pl.ds(0, chunk + 8), :], buf, sem).wait()

            for b in range(nbuf):
                dma_start(b, bufs[b], sems[b])

            def tinit(j, _):
                his_v[j] = n + 1
                return 0
            lax.fori_loop(0, tw, tinit, 0)

            def tbody(s_, carry):
                k_, lo_ = carry
                hi_ = pscal(ptr_v, s_ + 1)

                def t_yes(k__):
                    his_v[k__] = hi_
                    ids_v[k__] = s_
                    return k__ + 1

                k_ = lax.cond(hi_ > lo_, t_yes, lambda k__: k__, k_)
                return (k_, hi_)

            lax.fori_loop(0, seg_per_w, tbody, (jnp.int32(0), rlo))

            def process(c, buf_v, carry):
                klo, r, prev = carry[0], carry[1], carry[2]
                acc = tuple(carry[3:])
                base = chunk_base(c)
                clim = jnp.minimum(rlo + c * chunk + chunk, rhi)

                blo = jnp.int32(0)
                bhi = jnp.int32(tw)
                for _ in range(bsteps):
                    mid = (blo + bhi) // 2
                    gt = his_v[mid] > clim
                    live = blo < bhi
                    blo = jnp.where(jnp.logical_and(live, ~gt), mid + 1, blo)
                    bhi = jnp.where(jnp.logical_and(live, gt), mid, bhi)
                khi = blo

                def seg_body(k, st):
                    start, prev_ = st[0], st[1]
                    a = tuple(st[2:])
                    hi_k = his_v[k]
                    a = reduce_rows(buf_v, base, start, hi_k, a)
                    row = ids_v[k]

                    def zfill(j, _):
                        for q in range(kg):
                            m_v[j, pl.ds(_LANES * q, _LANES)] = zero
                        return 0
                    lax.fori_loop(prev_ + 1, row, zfill, 0)

                    for q in range(kg):
                        m_v[row, pl.ds(_LANES * q, _LANES)] = a[q]
                    return (hi_k, row) + tuple(neg for _ in range(kg))

                st = lax.fori_loop(klo, khi, seg_body, (r, prev) + acc)
                start, prev = st[0], st[1]
                acc = tuple(st[2:])
                acc = reduce_rows(buf_v, base, start, clim, acc)
                return (khi, jnp.maximum(clim, r), prev) + acc

            nh = (nch + nbuf - 1) // nbuf

            def round_body(h, carry):
                for b in range(nbuf):
                    c = h * nbuf + b
                    dma_wait(c, bufs[b], sems[b])
                    carry = process(c, bufs[b], carry)
                    dma_start(c + nbuf, bufs[b], sems[b])
                return carry

            init = (jnp.int32(0), rlo, jnp.int32(-1)) + tuple(
                neg for _ in range(kg))
            fin = lax.fori_loop(0, nh, round_body, init)
            prev = fin[2]

            def zfill_tail(j, _):
                for q in range(kg):
                    m_v[j, pl.ds(_LANES * q, _LANES)] = zero
                return 0
            lax.fori_loop(prev + 1, zlim, zfill_tail, 0)

            out_dma = pltpu.make_async_copy(
                m_v, m_hbm.at[pl.ds(s0, seg_per_w), :], semo)
            out_dma.start()
            out_dma.wait()

    f = pl.kernel(
        body,
        out_type=jax.ShapeDtypeStruct((nseg_pad, d), jnp.float32),
        mesh=mesh,
        scratch_types=(
            [
                pltpu.VMEM((seg_per_w + 24,), jnp.int32),
                pltpu.VMEM((seg_per_w, d), jnp.float32),
                pltpu.SMEM((tw,), jnp.int32),
                pltpu.SMEM((tw,), jnp.int32),
                pltpu.SemaphoreType.DMA,
            ]
            + [pltpu.VMEM((chunk + 8, d), jnp.float32)] * nbuf
            + [pltpu.SemaphoreType.DMA] * nbuf
        ),
    )
    return f(x_a, ptr_a)


def _tc_proj_one(m, W, b, prev, *, nseg, d, out_dim, blk):
    ngrid = nseg // blk
    has_prev = prev is not None

    def body(*refs):
        if has_prev:
            mm, ww, bb, pp, out = refs
        else:
            mm, ww, bb, out = refs
            pp = None
        i = pl.program_id(0)
        y = jnp.dot(mm[...], ww[...], preferred_element_type=jnp.float32)
        cand = jnp.max(y, axis=0, keepdims=True) + bb[...]
        cand = jnp.maximum(cand, 0.0)

        @pl.when(i == 0)
        def _():
            if has_prev:
                out[...] = jnp.maximum(cand, pp[...])
            else:
                out[...] = cand

        @pl.when(i > 0)
        def _():
            out[...] = jnp.maximum(out[...], cand)

    in_specs = [
        pl.BlockSpec((blk, d), lambda i: (i, 0)),
        pl.BlockSpec((d, out_dim), lambda i: (0, 0)),
        pl.BlockSpec((1, out_dim), lambda i: (0, 0)),
    ]
    args = [m, W, b.reshape(1, -1)]
    if has_prev:
        in_specs.append(pl.BlockSpec((1, out_dim), lambda i: (0, 0)))
        args.append(prev)
    return pl.pallas_call(
        body,
        grid=(ngrid,),
        in_specs=in_specs,
        out_specs=pl.BlockSpec((1, out_dim), lambda i: (0, 0)),
        out_shape=jax.ShapeDtypeStruct((1, out_dim), jnp.float32),
    )(*args)


def kernel(x_author, x_paper, ptr_author, ptr_paper,
           W_author, b_author, W_paper, b_paper):
    n, d = x_author.shape
    s = ptr_author.shape[0] - 1
    out_dim = W_author.shape[1]
    n_workers = 32
    seg_per_w = (-(-s // n_workers) + 7) // 8 * 8
    nseg_pad = n_workers * seg_per_w
    tw = seg_per_w + 32
    chunk = 128
    nbuf = 4

    pad = jnp.full((nseg_pad + 24 - (s + 1),), n, jnp.int32)
    ptr_a = jnp.concatenate([ptr_author.astype(jnp.int32), pad])
    ptr_p = jnp.concatenate([ptr_paper.astype(jnp.int32), pad])

    kw = dict(n=n, d=d, seg_per_w=seg_per_w, chunk=chunk,
              n_workers=n_workers, tw=tw, nseg_real=s, nbuf=nbuf)
    blk = 1000 if s % 1000 == 0 else 8

    m_a = _sc_segmax(x_author, ptr_a, **kw)
    m_p = _sc_segmax(x_paper, ptr_p, **kw)
    part = _tc_proj_one(m_a, W_author, b_author, None,
                        nseg=s, d=d, out_dim=out_dim, blk=blk)
    out = _tc_proj_one(m_p, W_paper, b_paper, part,
                       nseg=s, d=d, out_dim=out_dim, blk=blk)
    return out.reshape(out_dim)


# speedup vs baseline: 1.0522x; 1.0522x over previous
import jax
import jax.numpy as jnp
from jax import lax
from jax.experimental import pallas as pl
from jax.experimental.pallas import tpu as pltpu
from jax.experimental.pallas import tpu_sc as plsc

_LANES = 16


def _sc_segmax(x_a, x_p, ptr_a, ptr_p, *, n, d, seg_per_w, chunk,
               n_workers, tw, nseg_real, nbuf):
    nseg_pad = n_workers * seg_per_w
    kg = d // _LANES
    mesh = plsc.VectorSubcoreMesh(core_axis_name="c", subcore_axis_name="s")
    num_cores = 2
    bsteps = max(1, (tw - 1).bit_length())

    def pscal(ref, i):
        return ref[pl.ds(i, _LANES)][0]

    def body(xa_hbm, xp_hbm, pa_hbm, pp_hbm, ma_hbm, mp_hbm,
             ptr_v, m_v, his_v, ids_v, semo, *bufs_sems):
        bufs = bufs_sems[:nbuf]
        sems = bufs_sems[nbuf:]
        wid = lax.axis_index("s") * num_cores + lax.axis_index("c")
        s0 = pl.multiple_of(wid * seg_per_w, 8)
        neg = jnp.full((_LANES,), -jnp.inf, jnp.float32)
        zero = jnp.zeros((_LANES,), jnp.float32)
        zlim = jnp.clip(nseg_real - wid * seg_per_w, 0, seg_per_w)

        def reduce_rows(buf_v, base, g0, g1, acc):
            def rbody(g, a):
                idx = g - base
                return tuple(
                    jnp.maximum(a[q], buf_v[idx, pl.ds(_LANES * q, _LANES)])
                    for q in range(kg))
            return lax.fori_loop(g0, g1, rbody, acc)

        prev_out = None
        for x_hbm, p_hbm, m_hbm in ((xa_hbm, pa_hbm, ma_hbm),
                                    (xp_hbm, pp_hbm, mp_hbm)):
            pltpu.sync_copy(p_hbm.at[pl.ds(s0, seg_per_w + 24)], ptr_v)
            rlo = pscal(ptr_v, 0)
            rhi = pscal(ptr_v, seg_per_w)
            nch = jnp.maximum((rhi - rlo + chunk - 1) // chunk, 1)

            def chunk_base(c):
                clo = rlo + c * chunk
                return pl.multiple_of(
                    jnp.minimum(clo - clo % 8, n - (chunk + 8)), 8)

            def dma_start(c, buf, sem):
                @pl.when(c < nch)
                def _():
                    pltpu.make_async_copy(
                        x_hbm.at[pl.ds(chunk_base(c), chunk + 8), :], buf,
                        sem).start()

            def dma_wait(c, buf, sem):
                @pl.when(c < nch)
                def _():
                    pltpu.make_async_copy(
                        x_hbm.at[pl.ds(0, chunk + 8), :], buf, sem).wait()

            for b in range(nbuf):
                dma_start(b, bufs[b], sems[b])

            def tinit(j, _):
                his_v[j] = n + 1
                return 0
            lax.fori_loop(0, tw, tinit, 0)

            def tbody(s_, carry):
                k_, lo_ = carry
                hi_ = pscal(ptr_v, s_ + 1)

                def t_yes(k__):
                    his_v[k__] = hi_
                    ids_v[k__] = s_
                    return k__ + 1

                k_ = lax.cond(hi_ > lo_, t_yes, lambda k__: k__, k_)
                return (k_, hi_)

            lax.fori_loop(0, seg_per_w, tbody, (jnp.int32(0), rlo))

            if prev_out is not None:
                prev_out.wait()

            def process(c, buf_v, carry):
                klo, r, prev = carry[0], carry[1], carry[2]
                acc = tuple(carry[3:])
                base = chunk_base(c)
                clim = jnp.minimum(rlo + c * chunk + chunk, rhi)

                blo = jnp.int32(0)
                bhi = jnp.int32(tw)
                for _ in range(bsteps):
                    mid = (blo + bhi) // 2
                    gt = his_v[mid] > clim
                    live = blo < bhi
                    blo = jnp.where(jnp.logical_and(live, ~gt), mid + 1, blo)
                    bhi = jnp.where(jnp.logical_and(live, gt), mid, bhi)
                khi = blo

                def seg_body(k, st):
                    start, prev_ = st[0], st[1]
                    a = tuple(st[2:])
                    hi_k = his_v[k]
                    a = reduce_rows(buf_v, base, start, hi_k, a)
                    row = ids_v[k]

                    def zfill(j, _):
                        for q in range(kg):
                            m_v[j, pl.ds(_LANES * q, _LANES)] = zero
                        return 0
                    lax.fori_loop(prev_ + 1, row, zfill, 0)

                    for q in range(kg):
                        m_v[row, pl.ds(_LANES * q, _LANES)] = a[q]
                    return (hi_k, row) + tuple(neg for _ in range(kg))

                st = lax.fori_loop(klo, khi, seg_body, (r, prev) + acc)
                start, prev = st[0], st[1]
                acc = tuple(st[2:])
                acc = reduce_rows(buf_v, base, start, clim, acc)
                return (khi, jnp.maximum(clim, r), prev) + acc

            nh = (nch + nbuf - 1) // nbuf

            def round_body(h, carry):
                for b in range(nbuf):
                    c = h * nbuf + b
                    dma_wait(c, bufs[b], sems[b])
                    carry = process(c, bufs[b], carry)
                    dma_start(c + nbuf, bufs[b], sems[b])
                return carry

            init = (jnp.int32(0), rlo, jnp.int32(-1)) + tuple(
                neg for _ in range(kg))
            fin = lax.fori_loop(0, nh, round_body, init)
            prev = fin[2]

            def zfill_tail(j, _):
                for q in range(kg):
                    m_v[j, pl.ds(_LANES * q, _LANES)] = zero
                return 0
            lax.fori_loop(prev + 1, zlim, zfill_tail, 0)

            prev_out = pltpu.make_async_copy(
                m_v, m_hbm.at[pl.ds(s0, seg_per_w), :], semo)
            prev_out.start()
        prev_out.wait()

    f = pl.kernel(
        body,
        out_type=(
            jax.ShapeDtypeStruct((nseg_pad, d), jnp.float32),
            jax.ShapeDtypeStruct((nseg_pad, d), jnp.float32),
        ),
        mesh=mesh,
        scratch_types=(
            [
                pltpu.VMEM((seg_per_w + 24,), jnp.int32),
                pltpu.VMEM((seg_per_w, d), jnp.float32),
                pltpu.SMEM((tw,), jnp.int32),
                pltpu.SMEM((tw,), jnp.int32),
                pltpu.SemaphoreType.DMA,
            ]
            + [pltpu.VMEM((chunk + 8, d), jnp.float32)] * nbuf
            + [pltpu.SemaphoreType.DMA] * nbuf
        ),
    )
    return f(x_a, x_p, ptr_a, ptr_p)


def _tc_proj_reduce(m_a, m_p, W_a, b_a, W_p, b_p, *, nseg, d, out_dim, blk):
    ngrid = nseg // blk

    def body(ma, mp, wa, ba, wp, bp, out):
        i = pl.program_id(0)
        ya = jnp.dot(ma[...], wa[...], preferred_element_type=jnp.float32)
        yp = jnp.dot(mp[...], wp[...], preferred_element_type=jnp.float32)
        cand = jnp.maximum(
            jnp.max(ya, axis=0, keepdims=True) + ba[...],
            jnp.max(yp, axis=0, keepdims=True) + bp[...],
        )
        cand = jnp.maximum(cand, 0.0)

        @pl.when(i == 0)
        def _():
            out[...] = cand

        @pl.when(i > 0)
        def _():
            out[...] = jnp.maximum(out[...], cand)

    return pl.pallas_call(
        body,
        grid=(ngrid,),
        in_specs=[
            pl.BlockSpec((blk, d), lambda i: (i, 0)),
            pl.BlockSpec((blk, d), lambda i: (i, 0)),
            pl.BlockSpec((d, out_dim), lambda i: (0, 0)),
            pl.BlockSpec((1, out_dim), lambda i: (0, 0)),
            pl.BlockSpec((d, out_dim), lambda i: (0, 0)),
            pl.BlockSpec((1, out_dim), lambda i: (0, 0)),
        ],
        out_specs=pl.BlockSpec((1, out_dim), lambda i: (0, 0)),
        out_shape=jax.ShapeDtypeStruct((1, out_dim), jnp.float32),
    )(m_a, m_p, W_a, b_a.reshape(1, -1), W_p, b_p.reshape(1, -1))


def kernel(x_author, x_paper, ptr_author, ptr_paper,
           W_author, b_author, W_paper, b_paper):
    n, d = x_author.shape
    s = ptr_author.shape[0] - 1
    out_dim = W_author.shape[1]
    n_workers = 32
    seg_per_w = (-(-s // n_workers) + 7) // 8 * 8
    nseg_pad = n_workers * seg_per_w
    tw = seg_per_w + 32
    chunk = 288
    nbuf = 2

    pad = jnp.full((nseg_pad + 24 - (s + 1),), n, jnp.int32)
    ptr_a = jnp.concatenate([ptr_author.astype(jnp.int32), pad])
    ptr_p = jnp.concatenate([ptr_paper.astype(jnp.int32), pad])

    m_a, m_p = _sc_segmax(x_author, x_paper, ptr_a, ptr_p,
                          n=n, d=d, seg_per_w=seg_per_w, chunk=chunk,
                          n_workers=n_workers, tw=tw, nseg_real=s, nbuf=nbuf)

    blk = 1000 if s % 1000 == 0 else 8
    out = _tc_proj_reduce(m_a, m_p, W_author, b_author, W_paper, b_paper,
                          nseg=s, d=d, out_dim=out_dim, blk=blk)
    return out.reshape(out_dim)
